# baseline (speedup 1.0000x reference)
import functools

import jax
import jax.numpy as jnp
from jax import lax
from jax.experimental import pallas as pl
from jax.experimental.pallas import tpu as pltpu
from jax.experimental.pallas import tpu_sc as plsc

N = 10000
NP = 10240
E = 320000
CH = 128
EC = E // CH
NC = 2
NS = 16
PSUB = NP // NS
G = 4
CB = 156
NG = CB // G


def _mm_a_body(x_ref, wl_ref, wr_ref, p_ref, r_ref):
    xb = x_ref[...]
    p = jnp.dot(xb, wl_ref[...], preferred_element_type=jnp.float32)
    d = p.shape[-1] // 2
    p_ref[0] = p[:, :d]
    p_ref[1] = p[:, d:]
    r_ref[...] = jnp.dot(xb, wr_ref[...], preferred_element_type=jnp.float32)


def _mm_a(x, Wl, Wr):
    M, K = x.shape
    D = Wl.shape[1]
    blk = 1000
    return pl.pallas_call(
        _mm_a_body,
        grid=(M // blk,),
        in_specs=[
            pl.BlockSpec((blk, K), lambda i: (i, 0)),
            pl.BlockSpec((K, D), lambda i: (0, 0)),
            pl.BlockSpec((K, D), lambda i: (0, 0)),
        ],
        out_specs=[
            pl.BlockSpec((2, blk, D // 2), lambda i: (0, i, 0)),
            pl.BlockSpec((blk, D), lambda i: (i, 0)),
        ],
        out_shape=[
            jax.ShapeDtypeStruct((2, NP, D // 2), jnp.float32),
            jax.ShapeDtypeStruct((M, D), jnp.float32),
        ],
    )(x, Wl, Wr)


def _mm_b_body(agg_ref, deg_ref, b_ref, r_ref, wl_ref, wr_ref,
               p_ref, r2_ref):
    agg = jnp.concatenate([agg_ref[0], agg_ref[1]], axis=-1)
    deg = deg_ref[:, 0]
    inv = 1.0 / jnp.maximum(deg, 1.0)
    h = jnp.maximum(agg * inv[:, None] + b_ref[...] + r_ref[...], 0.0)
    p = jnp.dot(h, wl_ref[...], preferred_element_type=jnp.float32)
    d = p.shape[-1] // 2
    p_ref[0] = p[:, :d]
    p_ref[1] = p[:, d:]
    r2_ref[...] = jnp.dot(h, wr_ref[...], preferred_element_type=jnp.float32)


def _mm_b(aggp, deg2d, b, r, Wl, Wr):
    M, D = r.shape
    D2 = Wl.shape[1]
    blk = 1000
    Dh = D // 2
    return pl.pallas_call(
        _mm_b_body,
        grid=(M // blk,),
        in_specs=[
            pl.BlockSpec((2, blk, Dh), lambda i: (0, i, 0)),
            pl.BlockSpec((blk, 1), lambda i: (i, 0)),
            pl.BlockSpec((1, D), lambda i: (0, 0)),
            pl.BlockSpec((blk, D), lambda i: (i, 0)),
            pl.BlockSpec((D, D2), lambda i: (0, 0)),
            pl.BlockSpec((D, D2), lambda i: (0, 0)),
        ],
        out_specs=[
            pl.BlockSpec((2, blk, D2 // 2), lambda i: (0, i, 0)),
            pl.BlockSpec((blk, D2), lambda i: (i, 0)),
        ],
        out_shape=[
            jax.ShapeDtypeStruct((2, NP, D2 // 2), jnp.float32),
            jax.ShapeDtypeStruct((M, D2), jnp.float32),
        ],
    )(aggp, deg2d, b, r, Wl, Wr)


def _final_body(agg_ref, deg_ref, b_ref, r_ref, z_ref):
    agg = jnp.concatenate([agg_ref[0], agg_ref[1]], axis=-1)
    deg = deg_ref[:, 0]
    inv = 1.0 / jnp.maximum(deg, 1.0)
    z_ref[...] = agg * inv[:, None] + b_ref[...] + r_ref[...]


def _final(aggp, deg2d, b, r):
    M, D = r.shape
    blk = 1000
    Dh = D // 2
    return pl.pallas_call(
        _final_body,
        grid=(M // blk,),
        in_specs=[
            pl.BlockSpec((2, blk, Dh), lambda i: (0, i, 0)),
            pl.BlockSpec((blk, 1), lambda i: (i, 0)),
            pl.BlockSpec((1, D), lambda i: (0, 0)),
            pl.BlockSpec((blk, D), lambda i: (i, 0)),
        ],
        out_specs=pl.BlockSpec((blk, D), lambda i: (i, 0)),
        out_shape=jax.ShapeDtypeStruct((M, D), jnp.float32),
    )(aggp, deg2d, b, r)


def _make_sc_agg(Dh, with_deg):
    mesh = plsc.VectorSubcoreMesh(core_axis_name="c", subcore_axis_name="s")
    out_type = [jax.ShapeDtypeStruct((NC, NP, Dh), jnp.float32)]
    scratch = [
        pltpu.VMEM((CB + 1, CH), jnp.int32),
        pltpu.VMEM((CB + 1, CH), jnp.int32),
        pltpu.VMEM((NG, CH), jnp.int32),
        pltpu.VMEM((2, G * CH, Dh), jnp.float32),
        pltpu.VMEM_SHARED((NP, Dh), jnp.float32),
        pltpu.VMEM_SHARED((NP, Dh), jnp.float32),
        pltpu.SemaphoreType.DMA,
        pltpu.SemaphoreType.DMA,
        pltpu.SemaphoreType.DMA,
        pltpu.SemaphoreType.DMA,
    ]
    if with_deg:
        out_type.append(jax.ShapeDtypeStruct((NC, 1, NP), jnp.float32))
        scratch += [
            pltpu.VMEM((CH,), jnp.float32),
            pltpu.VMEM((PSUB,), jnp.float32),
            pltpu.VMEM_SHARED((NP,), jnp.float32),
            pltpu.SemaphoreType.DMA,
        ]

    @functools.partial(
        pl.kernel, mesh=mesh, out_type=out_type, scratch_types=scratch,
        compiler_params=pltpu.CompilerParams(use_tc_tiling_on_sc=False))
    def k(p_hbm, ei_hbm, *refs):
        if with_deg:
            (out_hbm, deg_hbm, src_v, dst_v, srco_v, rows_v, acc_sh, tbl_sh,
             sem_i, sem_g, sem_h, sem_s, ones_v, zero_v, deg_sh,
             sem_d) = refs
        else:
            (out_hbm, src_v, dst_v, srco_v, rows_v, acc_sh, tbl_sh,
             sem_i, sem_g, sem_h, sem_s) = refs
        c = lax.axis_index("c")
        s = lax.axis_index("s")
        base = s * CB + jnp.minimum(s, 4)
        extra = s < 4

        pltpu.async_copy(ei_hbm.at[0, pl.ds(base, CB)],
                         src_v.at[pl.ds(0, CB)], sem_i)
        pltpu.async_copy(ei_hbm.at[1, pl.ds(base, CB)],
                         dst_v.at[pl.ds(0, CB)], sem_i)
        pltpu.async_copy(p_hbm.at[pl.ds(c * NP + s * PSUB, PSUB)],
                         tbl_sh.at[pl.ds(s * PSUB, PSUB)], sem_i)
        @pl.when(extra)
        def _():
            pltpu.async_copy(ei_hbm.at[0, pl.ds(base + CB, 1)],
                             src_v.at[pl.ds(CB, 1)], sem_i)
            pltpu.async_copy(ei_hbm.at[1, pl.ds(base + CB, 1)],
                             dst_v.at[pl.ds(CB, 1)], sem_i)

        def zrow(i, carry):
            for jj in range(Dh // 16):
                rows_v[0, i, pl.ds(jj * 16, 16)] = jnp.zeros((16,),
                                                             jnp.float32)
            return carry
        lax.fori_loop(0, CH, zrow, 0)
        for kk in range(PSUB // CH):
            pltpu.sync_copy(rows_v.at[0, pl.ds(0, CH)],
                            acc_sh.at[pl.ds(s * PSUB + kk * CH, CH)])
        if with_deg:
            def zdeg(i, carry):
                zero_v[pl.ds(i * 16, 16)] = jnp.zeros((16,), jnp.float32)
                return carry
            lax.fori_loop(0, PSUB // 16, zdeg, 0)
            for jj in range(CH // 16):
                ones_v[pl.ds(jj * 16, 16)] = jnp.ones((16,), jnp.float32)
            pltpu.sync_copy(zero_v, deg_sh.at[pl.ds(s * PSUB, PSUB)])
        pltpu.make_async_copy(ei_hbm.at[0, pl.ds(0, CB)],
                              src_v.at[pl.ds(0, CB)], sem_i).wait()
        pltpu.make_async_copy(ei_hbm.at[0, pl.ds(0, CB)],
                              dst_v.at[pl.ds(0, CB)], sem_i).wait()
        pltpu.make_async_copy(p_hbm.at[pl.ds(0, PSUB)],
                              tbl_sh.at[pl.ds(0, PSUB)], sem_i).wait()
        @pl.when(extra)
        def _():
            for _x in range(2):
                pltpu.make_async_copy(ei_hbm.at[0, pl.ds(0, 1)],
                                      src_v.at[pl.ds(CB, 1)], sem_i).wait()
        cnp = c * NP

        def offs(n, carry):
            row = n * G + (G - 1)
            for jj in range(CH // 16):
                srco_v[n, pl.ds(jj * 16, 16)] = (
                    src_v[row, pl.ds(jj * 16, 16)] + cnp)
            return carry
        lax.fori_loop(0, NG, offs, 0)
        plsc.subcore_barrier()

        def g_start(n, ch, p, j):
            if j == G - 1:
                pltpu.async_copy(p_hbm.at[srco_v.at[n]],
                                 rows_v.at[p, pl.ds(j * CH, CH)], sem_h)
                return
            pltpu.async_copy(tbl_sh.at[src_v.at[ch]],
                             rows_v.at[p, pl.ds(j * CH, CH)], sem_g)

        def h_drain():
            pltpu.make_async_copy(p_hbm.at[pl.ds(0, CH)],
                                  rows_v.at[0, pl.ds(0, CH)], sem_h).wait()

        def g_drain():
            pltpu.make_async_copy(tbl_sh.at[pl.ds(0, CH)],
                                  rows_v.at[0, pl.ds(0, CH)], sem_g).wait()

        def s_start(ch, p, j):
            pltpu.async_copy(rows_v.at[p, pl.ds(j * CH, CH)],
                             acc_sh.at[dst_v.at[ch]], sem_s, add=True)

        def s_drain():
            pltpu.make_async_copy(rows_v.at[0, pl.ds(0, CH)],
                                  acc_sh.at[pl.ds(0, CH)], sem_s).wait()

        def d_start(ch):
            pltpu.async_copy(ones_v, deg_sh.at[dst_v.at[ch]], sem_d,
                             add=True)

        def d_drain():
            pltpu.make_async_copy(ones_v, deg_sh.at[pl.ds(0, CH)],
                                  sem_d).wait()

        for j in range(G):
            g_start(0, j, 0, j)

        def grp(n, carry):
            p = lax.rem(n, 2)
            for j in range(G - 1):
                g_drain()
            h_drain()
            @pl.when(n >= 1)
            def _():
                for j in range(G):
                    s_drain()
                if with_deg:
                    for j in range(G):
                        d_drain()
            @pl.when(n + 1 < NG)
            def _():
                for j in range(G):
                    g_start(n + 1, (n + 1) * G + j, 1 - p, j)
            for j in range(G):
                s_start(n * G + j, p, j)
            if with_deg:
                for j in range(G):
                    d_start(n * G + j)
            return carry
        lax.fori_loop(0, NG, grp, 0)
        for j in range(G):
            s_drain()
        if with_deg:
            for j in range(G):
                d_drain()
        @pl.when(extra)
        def _():
            g_start(0, CB, 0, 0)
            g_drain()
            s_start(CB, 0, 0)
            s_drain()
            if with_deg:
                d_start(CB)
                d_drain()
        plsc.subcore_barrier()

        pltpu.sync_copy(acc_sh.at[pl.ds(s * PSUB, PSUB)],
                        out_hbm.at[c, pl.ds(s * PSUB, PSUB)])
        if with_deg:
            pltpu.sync_copy(deg_sh.at[pl.ds(s * PSUB, PSUB)],
                            deg_hbm.at[c, 0, pl.ds(s * PSUB, PSUB)])

    return k


_sc_agg_cache = {}


def _sc_agg_call(Dh, with_deg, *args):
    key = (Dh, with_deg)
    if key not in _sc_agg_cache:
        _sc_agg_cache[key] = _make_sc_agg(Dh, with_deg)
    return _sc_agg_cache[key](*args)


def _impl(x, edge_index, Wl1, bl1, Wr1, Wl2, bl2, Wr2):
    ei = edge_index.astype(jnp.int32).reshape(2, EC, CH)

    p1s, r1 = _mm_a(x, Wl1, Wr1)
    agg1p, degp = _sc_agg_call(32, True, p1s.reshape(NC * NP, 32), ei)
    deg2d = degp[0].reshape(NP, 1)
    p2s, r2 = _mm_b(agg1p, deg2d, bl1.reshape(1, -1), r1, Wl2, Wr2)
    (agg2p,) = _sc_agg_call(16, False, p2s.reshape(NC * NP, 16), ei)
    z = _final(agg2p, deg2d, bl2.reshape(1, -1), r2)
    return z


kernel = jax.jit(_impl)


# speedup vs baseline: 1.0756x; 1.0756x over previous
import functools

import jax
import jax.numpy as jnp
from jax import lax
from jax.experimental import pallas as pl
from jax.experimental.pallas import tpu as pltpu
from jax.experimental.pallas import tpu_sc as plsc

N = 10000
NP = 10240
E = 320000
CH = 128
EC = E // CH
NC = 2
NS = 16
PSUB = NP // NS
G = 4
CB = 156
NG = CB // G
NG2 = 20


def _mm_a_body(x_ref, wl_ref, wr_ref, p_ref, r_ref):
    xb = x_ref[...]
    p = jnp.dot(xb, wl_ref[...], preferred_element_type=jnp.float32)
    d = p.shape[-1] // 2
    p_ref[0] = p[:, :d]
    p_ref[1] = p[:, d:]
    r_ref[...] = jnp.dot(xb, wr_ref[...], preferred_element_type=jnp.float32)


def _mm_a(x, Wl, Wr):
    M, K = x.shape
    D = Wl.shape[1]
    blk = 1000
    return pl.pallas_call(
        _mm_a_body,
        grid=(M // blk,),
        in_specs=[
            pl.BlockSpec((blk, K), lambda i: (i, 0)),
            pl.BlockSpec((K, D), lambda i: (0, 0)),
            pl.BlockSpec((K, D), lambda i: (0, 0)),
        ],
        out_specs=[
            pl.BlockSpec((2, blk, D // 2), lambda i: (0, i, 0)),
            pl.BlockSpec((blk, D), lambda i: (i, 0)),
        ],
        out_shape=[
            jax.ShapeDtypeStruct((2, NP, D // 2), jnp.float32),
            jax.ShapeDtypeStruct((M, D), jnp.float32),
        ],
    )(x, Wl, Wr)


def _mm_b_body(agg_ref, deg_ref, b_ref, r_ref, wl_ref, wr_ref,
               p_ref, r2_ref):
    agg = jnp.concatenate([agg_ref[0], agg_ref[1]], axis=-1)
    deg = deg_ref[:, 0]
    inv = 1.0 / jnp.maximum(deg, 1.0)
    h = jnp.maximum(agg * inv[:, None] + b_ref[...] + r_ref[...], 0.0)
    p = jnp.dot(h, wl_ref[...], preferred_element_type=jnp.float32)
    d = p.shape[-1] // 2
    p_ref[0] = p[:, :d]
    p_ref[1] = p[:, d:]
    r2_ref[...] = jnp.dot(h, wr_ref[...], preferred_element_type=jnp.float32)


def _mm_b(aggp, deg2d, b, r, Wl, Wr):
    M, D = r.shape
    D2 = Wl.shape[1]
    blk = 1000
    Dh = D // 2
    return pl.pallas_call(
        _mm_b_body,
        grid=(M // blk,),
        in_specs=[
            pl.BlockSpec((2, blk, Dh), lambda i: (0, i, 0)),
            pl.BlockSpec((blk, 1), lambda i: (i, 0)),
            pl.BlockSpec((1, D), lambda i: (0, 0)),
            pl.BlockSpec((blk, D), lambda i: (i, 0)),
            pl.BlockSpec((D, D2), lambda i: (0, 0)),
            pl.BlockSpec((D, D2), lambda i: (0, 0)),
        ],
        out_specs=[
            pl.BlockSpec((2, blk, D2 // 2), lambda i: (0, i, 0)),
            pl.BlockSpec((blk, D2), lambda i: (i, 0)),
        ],
        out_shape=[
            jax.ShapeDtypeStruct((2, NP, D2 // 2), jnp.float32),
            jax.ShapeDtypeStruct((M, D2), jnp.float32),
        ],
    )(aggp, deg2d, b, r, Wl, Wr)


def _final_body(agg_ref, deg_ref, b_ref, r_ref, z_ref):
    agg = jnp.concatenate([agg_ref[0], agg_ref[1]], axis=-1)
    deg = deg_ref[:, 0]
    inv = 1.0 / jnp.maximum(deg, 1.0)
    z_ref[...] = agg * inv[:, None] + b_ref[...] + r_ref[...]


def _final(aggp, deg2d, b, r):
    M, D = r.shape
    blk = 1000
    Dh = D // 2
    return pl.pallas_call(
        _final_body,
        grid=(M // blk,),
        in_specs=[
            pl.BlockSpec((2, blk, Dh), lambda i: (0, i, 0)),
            pl.BlockSpec((blk, 1), lambda i: (i, 0)),
            pl.BlockSpec((1, D), lambda i: (0, 0)),
            pl.BlockSpec((blk, D), lambda i: (i, 0)),
        ],
        out_specs=pl.BlockSpec((blk, D), lambda i: (i, 0)),
        out_shape=jax.ShapeDtypeStruct((M, D), jnp.float32),
    )(aggp, deg2d, b, r)


def _make_sc_agg(Dh, with_deg):
    mesh = plsc.VectorSubcoreMesh(core_axis_name="c", subcore_axis_name="s")
    out_type = [jax.ShapeDtypeStruct((NC, NP, Dh), jnp.float32)]
    scratch = [
        pltpu.VMEM((CB + 1, CH), jnp.int32),
        pltpu.VMEM((CB + 1, CH), jnp.int32),
        pltpu.VMEM((2, G * CH, Dh), jnp.float32),
        pltpu.VMEM_SHARED((NP, Dh), jnp.float32),
        pltpu.VMEM_SHARED((NP, Dh), jnp.float32),
        pltpu.SemaphoreType.DMA,
        pltpu.SemaphoreType.DMA,
        pltpu.SemaphoreType.DMA,
    ]
    if with_deg:
        out_type.append(jax.ShapeDtypeStruct((NC, 1, NP), jnp.float32))
        scratch += [
            pltpu.VMEM((CH,), jnp.float32),
            pltpu.VMEM((PSUB,), jnp.float32),
            pltpu.VMEM_SHARED((NP,), jnp.float32),
            pltpu.SemaphoreType.DMA,
        ]

    @functools.partial(
        pl.kernel, mesh=mesh, out_type=out_type, scratch_types=scratch,
        compiler_params=pltpu.CompilerParams(use_tc_tiling_on_sc=False))
    def k(p_hbm, ei_hbm, *refs):
        if with_deg:
            (out_hbm, deg_hbm, src_v, dst_v, rows_v, acc_sh, tbl_sh,
             sem_i, sem_g, sem_s, ones_v, zero_v, deg_sh, sem_d) = refs
        else:
            (out_hbm, src_v, dst_v, rows_v, acc_sh, tbl_sh,
             sem_i, sem_g, sem_s) = refs
        c = lax.axis_index("c")
        s = lax.axis_index("s")
        base = s * CB + jnp.minimum(s, 4)
        extra = s < 4

        pltpu.async_copy(ei_hbm.at[0, pl.ds(base, CB)],
                         src_v.at[pl.ds(0, CB)], sem_i)
        pltpu.async_copy(ei_hbm.at[1, pl.ds(base, CB)],
                         dst_v.at[pl.ds(0, CB)], sem_i)
        pltpu.async_copy(p_hbm.at[c, pl.ds(s * PSUB, PSUB)],
                         tbl_sh.at[pl.ds(s * PSUB, PSUB)], sem_i)
        @pl.when(extra)
        def _():
            pltpu.async_copy(ei_hbm.at[0, pl.ds(base + CB, 1)],
                             src_v.at[pl.ds(CB, 1)], sem_i)
            pltpu.async_copy(ei_hbm.at[1, pl.ds(base + CB, 1)],
                             dst_v.at[pl.ds(CB, 1)], sem_i)

        def zrow(i, carry):
            for jj in range(Dh // 16):
                rows_v[0, i, pl.ds(jj * 16, 16)] = jnp.zeros((16,),
                                                             jnp.float32)
            return carry
        lax.fori_loop(0, CH, zrow, 0)
        for kk in range(PSUB // CH):
            pltpu.sync_copy(rows_v.at[0, pl.ds(0, CH)],
                            acc_sh.at[pl.ds(s * PSUB + kk * CH, CH)])
        if with_deg:
            def zdeg(i, carry):
                zero_v[pl.ds(i * 16, 16)] = jnp.zeros((16,), jnp.float32)
                return carry
            lax.fori_loop(0, PSUB // 16, zdeg, 0)
            for jj in range(CH // 16):
                ones_v[pl.ds(jj * 16, 16)] = jnp.ones((16,), jnp.float32)
            pltpu.sync_copy(zero_v, deg_sh.at[pl.ds(s * PSUB, PSUB)])
        pltpu.make_async_copy(ei_hbm.at[0, pl.ds(0, CB)],
                              src_v.at[pl.ds(0, CB)], sem_i).wait()
        pltpu.make_async_copy(ei_hbm.at[0, pl.ds(0, CB)],
                              dst_v.at[pl.ds(0, CB)], sem_i).wait()
        pltpu.make_async_copy(p_hbm.at[0, pl.ds(0, PSUB)],
                              tbl_sh.at[pl.ds(0, PSUB)], sem_i).wait()
        @pl.when(extra)
        def _():
            for _x in range(2):
                pltpu.make_async_copy(ei_hbm.at[0, pl.ds(0, 1)],
                                      src_v.at[pl.ds(CB, 1)], sem_i).wait()
        plsc.subcore_barrier()

        def g_start(ch, p, j):
            pltpu.async_copy(tbl_sh.at[src_v.at[ch]],
                             rows_v.at[p, pl.ds(j * CH, CH)], sem_g)

        def g_drain():
            pltpu.make_async_copy(tbl_sh.at[pl.ds(0, CH)],
                                  rows_v.at[0, pl.ds(0, CH)], sem_g).wait()

        def s_start(ch, p, j):
            pltpu.async_copy(rows_v.at[p, pl.ds(j * CH, CH)],
                             acc_sh.at[dst_v.at[ch]], sem_s, add=True)

        def s_drain():
            pltpu.make_async_copy(rows_v.at[0, pl.ds(0, CH)],
                                  acc_sh.at[pl.ds(0, CH)], sem_s).wait()

        def d_start(ch):
            pltpu.async_copy(ones_v, deg_sh.at[dst_v.at[ch]], sem_d,
                             add=True)

        def d_drain():
            pltpu.make_async_copy(ones_v, deg_sh.at[pl.ds(0, CH)],
                                  sem_d).wait()

        def fire_deg(n):
            return (c == 0) == (n < NG2)

        for j in range(G):
            g_start(j, 0, j)

        def grp(n, carry):
            p = lax.rem(n, 2)
            for j in range(G):
                g_drain()
            @pl.when(n >= 1)
            def _():
                for j in range(G):
                    s_drain()
            if with_deg:
                @pl.when((n >= 1) & fire_deg(n - 1))
                def _():
                    for j in range(G):
                        d_drain()
            @pl.when(n + 1 < NG)
            def _():
                for j in range(G):
                    g_start((n + 1) * G + j, 1 - p, j)
            for j in range(G):
                s_start(n * G + j, p, j)
            if with_deg:
                @pl.when(fire_deg(n))
                def _():
                    for j in range(G):
                        d_start(n * G + j)
            return carry
        lax.fori_loop(0, NG, grp, 0)
        for j in range(G):
            s_drain()
        if with_deg:
            @pl.when(fire_deg(NG - 1))
            def _():
                for j in range(G):
                    d_drain()
        @pl.when(extra)
        def _():
            g_start(CB, 0, 0)
            g_drain()
            s_start(CB, 0, 0)
            s_drain()
        if with_deg:
            @pl.when(extra & (c == 0))
            def _():
                d_start(CB)
                d_drain()
        plsc.subcore_barrier()

        pltpu.sync_copy(acc_sh.at[pl.ds(s * PSUB, PSUB)],
                        out_hbm.at[c, pl.ds(s * PSUB, PSUB)])
        if with_deg:
            pltpu.sync_copy(deg_sh.at[pl.ds(s * PSUB, PSUB)],
                            deg_hbm.at[c, 0, pl.ds(s * PSUB, PSUB)])

    return k


_sc_agg_cache = {}


def _sc_agg_call(Dh, with_deg, *args):
    key = (Dh, with_deg)
    if key not in _sc_agg_cache:
        _sc_agg_cache[key] = _make_sc_agg(Dh, with_deg)
    return _sc_agg_cache[key](*args)


def _impl(x, edge_index, Wl1, bl1, Wr1, Wl2, bl2, Wr2):
    ei = edge_index.astype(jnp.int32).reshape(2, EC, CH)

    p1s, r1 = _mm_a(x, Wl1, Wr1)
    agg1p, degp = _sc_agg_call(32, True, p1s, ei)
    deg2d = (degp[0] + degp[1]).reshape(NP, 1)
    p2s, r2 = _mm_b(agg1p, deg2d, bl1.reshape(1, -1), r1, Wl2, Wr2)
    (agg2p,) = _sc_agg_call(16, False, p2s, ei)
    z = _final(agg2p, deg2d, bl2.reshape(1, -1), r2)
    return z


kernel = jax.jit(_impl)


# speedup vs baseline: 1.0870x; 1.0106x over previous
import functools

import jax
import jax.numpy as jnp
from jax import lax
from jax.experimental import pallas as pl
from jax.experimental.pallas import tpu as pltpu
from jax.experimental.pallas import tpu_sc as plsc

N = 10000
NP = 10240
E = 320000
CH = 128
EC = E // CH
NC = 2
NS = 16
PSUB = NP // NS
G = 4
CB = 156
NG = CB // G
NG2 = 20


def _mm_a_body(x_ref, wl_ref, wr_ref, p_ref, r_ref):
    xb = x_ref[...]
    p = jnp.dot(xb, wl_ref[...], preferred_element_type=jnp.float32)
    d = p.shape[-1] // 2
    p_ref[0] = p[:, :d]
    p_ref[1] = p[:, d:]
    r_ref[...] = jnp.dot(xb, wr_ref[...], preferred_element_type=jnp.float32)


def _mm_a(x, Wl, Wr):
    M, K = x.shape
    D = Wl.shape[1]
    blk = 1000
    return pl.pallas_call(
        _mm_a_body,
        grid=(M // blk,),
        in_specs=[
            pl.BlockSpec((blk, K), lambda i: (i, 0)),
            pl.BlockSpec((K, D), lambda i: (0, 0)),
            pl.BlockSpec((K, D), lambda i: (0, 0)),
        ],
        out_specs=[
            pl.BlockSpec((2, blk, D // 2), lambda i: (0, i, 0)),
            pl.BlockSpec((blk, D), lambda i: (i, 0)),
        ],
        out_shape=[
            jax.ShapeDtypeStruct((2, NP, D // 2), jnp.float32),
            jax.ShapeDtypeStruct((M, D), jnp.float32),
        ],
    )(x, Wl, Wr)


def _mm_b_body(agg_ref, deg_ref, b_ref, r_ref, wl_ref, p_ref):
    agg = jnp.concatenate([agg_ref[0], agg_ref[1]], axis=-1)
    deg = deg_ref[:, 0]
    inv = 1.0 / jnp.maximum(deg, 1.0)
    h = jnp.maximum(agg * inv[:, None] + b_ref[...] + r_ref[...], 0.0)
    p = jnp.dot(h, wl_ref[...], preferred_element_type=jnp.float32)
    d = p.shape[-1] // 2
    p_ref[0] = p[:, :d]
    p_ref[1] = p[:, d:]


def _mm_r2_body(agg_ref, deg_ref, b_ref, r_ref, wr_ref, r2_ref):
    agg = jnp.concatenate([agg_ref[0], agg_ref[1]], axis=-1)
    deg = deg_ref[:, 0]
    inv = 1.0 / jnp.maximum(deg, 1.0)
    h = jnp.maximum(agg * inv[:, None] + b_ref[...] + r_ref[...], 0.0)
    r2_ref[...] = jnp.dot(h, wr_ref[...], preferred_element_type=jnp.float32)


def _mm_r2(aggp, deg2d, b, r, Wr):
    M, D = r.shape
    D2 = Wr.shape[1]
    blk = 1000
    Dh = D // 2
    return pl.pallas_call(
        _mm_r2_body,
        grid=(M // blk,),
        in_specs=[
            pl.BlockSpec((2, blk, Dh), lambda i: (0, i, 0)),
            pl.BlockSpec((blk, 1), lambda i: (i, 0)),
            pl.BlockSpec((1, D), lambda i: (0, 0)),
            pl.BlockSpec((blk, D), lambda i: (i, 0)),
            pl.BlockSpec((D, D2), lambda i: (0, 0)),
        ],
        out_specs=pl.BlockSpec((blk, D2), lambda i: (i, 0)),
        out_shape=jax.ShapeDtypeStruct((M, D2), jnp.float32),
    )(aggp, deg2d, b, r, Wr)


def _mm_b(aggp, deg2d, b, r, Wl):
    M, D = r.shape
    D2 = Wl.shape[1]
    blk = 1000
    Dh = D // 2
    return pl.pallas_call(
        _mm_b_body,
        grid=(M // blk,),
        in_specs=[
            pl.BlockSpec((2, blk, Dh), lambda i: (0, i, 0)),
            pl.BlockSpec((blk, 1), lambda i: (i, 0)),
            pl.BlockSpec((1, D), lambda i: (0, 0)),
            pl.BlockSpec((blk, D), lambda i: (i, 0)),
            pl.BlockSpec((D, D2), lambda i: (0, 0)),
        ],
        out_specs=pl.BlockSpec((2, blk, D2 // 2), lambda i: (0, i, 0)),
        out_shape=jax.ShapeDtypeStruct((2, NP, D2 // 2), jnp.float32),
    )(aggp, deg2d, b, r, Wl)


def _final_body(agg_ref, deg_ref, b_ref, r_ref, z_ref):
    agg = jnp.concatenate([agg_ref[0], agg_ref[1]], axis=-1)
    deg = deg_ref[:, 0]
    inv = 1.0 / jnp.maximum(deg, 1.0)
    z_ref[...] = agg * inv[:, None] + b_ref[...] + r_ref[...]


def _final(aggp, deg2d, b, r):
    M, D = r.shape
    blk = 1000
    Dh = D // 2
    return pl.pallas_call(
        _final_body,
        grid=(M // blk,),
        in_specs=[
            pl.BlockSpec((2, blk, Dh), lambda i: (0, i, 0)),
            pl.BlockSpec((blk, 1), lambda i: (i, 0)),
            pl.BlockSpec((1, D), lambda i: (0, 0)),
            pl.BlockSpec((blk, D), lambda i: (i, 0)),
        ],
        out_specs=pl.BlockSpec((blk, D), lambda i: (i, 0)),
        out_shape=jax.ShapeDtypeStruct((M, D), jnp.float32),
    )(aggp, deg2d, b, r)


def _make_sc_agg(Dh, with_deg):
    mesh = plsc.VectorSubcoreMesh(core_axis_name="c", subcore_axis_name="s")
    out_type = [jax.ShapeDtypeStruct((NC, NP, Dh), jnp.float32)]
    scratch = [
        pltpu.VMEM((CB + 1, CH), jnp.int32),
        pltpu.VMEM((CB + 1, CH), jnp.int32),
        pltpu.VMEM((2, G * CH, Dh), jnp.float32),
        pltpu.VMEM_SHARED((NP, Dh), jnp.float32),
        pltpu.VMEM_SHARED((NP, Dh), jnp.float32),
        pltpu.SemaphoreType.DMA,
        pltpu.SemaphoreType.DMA,
        pltpu.SemaphoreType.DMA,
    ]
    if with_deg:
        out_type.append(jax.ShapeDtypeStruct((NC, 1, NP), jnp.float32))
        scratch += [
            pltpu.VMEM((CH,), jnp.float32),
            pltpu.VMEM((PSUB,), jnp.float32),
            pltpu.VMEM_SHARED((NP,), jnp.float32),
            pltpu.SemaphoreType.DMA,
        ]

    @functools.partial(
        pl.kernel, mesh=mesh, out_type=out_type, scratch_types=scratch,
        compiler_params=pltpu.CompilerParams(use_tc_tiling_on_sc=False))
    def k(p_hbm, ei_hbm, *refs):
        if with_deg:
            (out_hbm, deg_hbm, src_v, dst_v, rows_v, acc_sh, tbl_sh,
             sem_i, sem_g, sem_s, ones_v, zero_v, deg_sh, sem_d) = refs
        else:
            (out_hbm, src_v, dst_v, rows_v, acc_sh, tbl_sh,
             sem_i, sem_g, sem_s) = refs
        c = lax.axis_index("c")
        s = lax.axis_index("s")
        base = s * CB + jnp.minimum(s, 4)
        extra = s < 4

        pltpu.async_copy(ei_hbm.at[0, pl.ds(base, CB)],
                         src_v.at[pl.ds(0, CB)], sem_i)
        pltpu.async_copy(ei_hbm.at[1, pl.ds(base, CB)],
                         dst_v.at[pl.ds(0, CB)], sem_i)
        pltpu.async_copy(p_hbm.at[c, pl.ds(s * PSUB, PSUB)],
                         tbl_sh.at[pl.ds(s * PSUB, PSUB)], sem_i)
        @pl.when(extra)
        def _():
            pltpu.async_copy(ei_hbm.at[0, pl.ds(base + CB, 1)],
                             src_v.at[pl.ds(CB, 1)], sem_i)
            pltpu.async_copy(ei_hbm.at[1, pl.ds(base + CB, 1)],
                             dst_v.at[pl.ds(CB, 1)], sem_i)

        def zrow(i, carry):
            for jj in range(Dh // 16):
                rows_v[0, i, pl.ds(jj * 16, 16)] = jnp.zeros((16,),
                                                             jnp.float32)
            return carry
        lax.fori_loop(0, CH, zrow, 0)
        for kk in range(PSUB // CH):
            pltpu.sync_copy(rows_v.at[0, pl.ds(0, CH)],
                            acc_sh.at[pl.ds(s * PSUB + kk * CH, CH)])
        if with_deg:
            def zdeg(i, carry):
                zero_v[pl.ds(i * 16, 16)] = jnp.zeros((16,), jnp.float32)
                return carry
            lax.fori_loop(0, PSUB // 16, zdeg, 0)
            for jj in range(CH // 16):
                ones_v[pl.ds(jj * 16, 16)] = jnp.ones((16,), jnp.float32)
            pltpu.sync_copy(zero_v, deg_sh.at[pl.ds(s * PSUB, PSUB)])
        pltpu.make_async_copy(ei_hbm.at[0, pl.ds(0, CB)],
                              src_v.at[pl.ds(0, CB)], sem_i).wait()
        pltpu.make_async_copy(ei_hbm.at[0, pl.ds(0, CB)],
                              dst_v.at[pl.ds(0, CB)], sem_i).wait()
        pltpu.make_async_copy(p_hbm.at[0, pl.ds(0, PSUB)],
                              tbl_sh.at[pl.ds(0, PSUB)], sem_i).wait()
        @pl.when(extra)
        def _():
            for _x in range(2):
                pltpu.make_async_copy(ei_hbm.at[0, pl.ds(0, 1)],
                                      src_v.at[pl.ds(CB, 1)], sem_i).wait()
        plsc.subcore_barrier()

        def g_start(ch, p, j):
            pltpu.async_copy(tbl_sh.at[src_v.at[ch]],
                             rows_v.at[p, pl.ds(j * CH, CH)], sem_g)

        def g_drain():
            pltpu.make_async_copy(tbl_sh.at[pl.ds(0, CH)],
                                  rows_v.at[0, pl.ds(0, CH)], sem_g).wait()

        def s_start(ch, p, j):
            pltpu.async_copy(rows_v.at[p, pl.ds(j * CH, CH)],
                             acc_sh.at[dst_v.at[ch]], sem_s, add=True)

        def s_drain():
            pltpu.make_async_copy(rows_v.at[0, pl.ds(0, CH)],
                                  acc_sh.at[pl.ds(0, CH)], sem_s).wait()

        def d_start(ch):
            pltpu.async_copy(ones_v, deg_sh.at[dst_v.at[ch]], sem_d,
                             add=True)

        def d_drain():
            pltpu.make_async_copy(ones_v, deg_sh.at[pl.ds(0, CH)],
                                  sem_d).wait()

        def fire_deg(n):
            return (c == 0) == (n < NG2)

        for j in range(G):
            g_start(j, 0, j)

        def grp(n, carry):
            p = lax.rem(n, 2)
            for j in range(G):
                g_drain()
            @pl.when(n >= 1)
            def _():
                for j in range(G):
                    s_drain()
            if with_deg:
                @pl.when((n >= 1) & fire_deg(n - 1))
                def _():
                    for j in range(G):
                        d_drain()
            @pl.when(n + 1 < NG)
            def _():
                for j in range(G):
                    g_start((n + 1) * G + j, 1 - p, j)
            for j in range(G):
                s_start(n * G + j, p, j)
            if with_deg:
                @pl.when(fire_deg(n))
                def _():
                    for j in range(G):
                        d_start(n * G + j)
            return carry
        lax.fori_loop(0, NG, grp, 0)
        for j in range(G):
            s_drain()
        if with_deg:
            @pl.when(fire_deg(NG - 1))
            def _():
                for j in range(G):
                    d_drain()
        @pl.when(extra)
        def _():
            g_start(CB, 0, 0)
            g_drain()
            s_start(CB, 0, 0)
            s_drain()
        if with_deg:
            @pl.when(extra & (c == 0))
            def _():
                d_start(CB)
                d_drain()
        plsc.subcore_barrier()

        pltpu.sync_copy(acc_sh.at[pl.ds(s * PSUB, PSUB)],
                        out_hbm.at[c, pl.ds(s * PSUB, PSUB)])
        if with_deg:
            pltpu.sync_copy(deg_sh.at[pl.ds(s * PSUB, PSUB)],
                            deg_hbm.at[c, 0, pl.ds(s * PSUB, PSUB)])

    return k


_sc_agg_cache = {}


def _sc_agg_call(Dh, with_deg, *args):
    key = (Dh, with_deg)
    if key not in _sc_agg_cache:
        _sc_agg_cache[key] = _make_sc_agg(Dh, with_deg)
    return _sc_agg_cache[key](*args)


def _impl(x, edge_index, Wl1, bl1, Wr1, Wl2, bl2, Wr2):
    ei = edge_index.astype(jnp.int32).reshape(2, EC, CH)

    p1s, r1 = _mm_a(x, Wl1, Wr1)
    agg1p, degp = _sc_agg_call(32, True, p1s, ei)
    deg2d = (degp[0] + degp[1]).reshape(NP, 1)
    p2s = _mm_b(agg1p, deg2d, bl1.reshape(1, -1), r1, Wl2)
    (agg2p,) = _sc_agg_call(16, False, p2s, ei)
    r2 = _mm_r2(agg1p, deg2d, bl1.reshape(1, -1), r1, Wr2)
    z = _final(agg2p, deg2d, bl2.reshape(1, -1), r2)
    return z


kernel = jax.jit(_impl)


# speedup vs baseline: 1.0897x; 1.0025x over previous
import functools

import jax
import jax.numpy as jnp
from jax import lax
from jax.experimental import pallas as pl
from jax.experimental.pallas import tpu as pltpu
from jax.experimental.pallas import tpu_sc as plsc

N = 10000
NP = 10240
E = 320000
CH = 128
EC = E // CH
NC = 2
NS = 16
PSUB = NP // NS
G = 4
CB = 156
NG = CB // G
NG2 = 20


def _mm_a_body(x_ref, wl_ref, p_ref):
    xb = x_ref[...]
    p = jnp.dot(xb, wl_ref[...], preferred_element_type=jnp.float32)
    d = p.shape[-1] // 2
    p_ref[0] = p[:, :d]
    p_ref[1] = p[:, d:]


def _mm_r1_body(x_ref, wr_ref, r_ref):
    r_ref[...] = jnp.dot(x_ref[...], wr_ref[...],
                         preferred_element_type=jnp.float32)


def _mm_r1(x, Wr):
    M, K = x.shape
    D = Wr.shape[1]
    blk = 1000
    return pl.pallas_call(
        _mm_r1_body,
        grid=(M // blk,),
        in_specs=[
            pl.BlockSpec((blk, K), lambda i: (i, 0)),
            pl.BlockSpec((K, D), lambda i: (0, 0)),
        ],
        out_specs=pl.BlockSpec((blk, D), lambda i: (i, 0)),
        out_shape=jax.ShapeDtypeStruct((M, D), jnp.float32),
    )(x, Wr)


def _mm_a(x, Wl):
    M, K = x.shape
    D = Wl.shape[1]
    blk = 1000
    return pl.pallas_call(
        _mm_a_body,
        grid=(M // blk,),
        in_specs=[
            pl.BlockSpec((blk, K), lambda i: (i, 0)),
            pl.BlockSpec((K, D), lambda i: (0, 0)),
        ],
        out_specs=pl.BlockSpec((2, blk, D // 2), lambda i: (0, i, 0)),
        out_shape=jax.ShapeDtypeStruct((2, NP, D // 2), jnp.float32),
    )(x, Wl)


def _mm_b_body(agg_ref, deg_ref, b_ref, r_ref, wl_ref, p_ref):
    agg = jnp.concatenate([agg_ref[0], agg_ref[1]], axis=-1)
    deg = deg_ref[:, 0]
    inv = 1.0 / jnp.maximum(deg, 1.0)
    h = jnp.maximum(agg * inv[:, None] + b_ref[...] + r_ref[...], 0.0)
    p = jnp.dot(h, wl_ref[...], preferred_element_type=jnp.float32)
    d = p.shape[-1] // 2
    p_ref[0] = p[:, :d]
    p_ref[1] = p[:, d:]


def _mm_r2_body(agg_ref, deg_ref, b_ref, r_ref, wr_ref, r2_ref):
    agg = jnp.concatenate([agg_ref[0], agg_ref[1]], axis=-1)
    deg = deg_ref[:, 0]
    inv = 1.0 / jnp.maximum(deg, 1.0)
    h = jnp.maximum(agg * inv[:, None] + b_ref[...] + r_ref[...], 0.0)
    r2_ref[...] = jnp.dot(h, wr_ref[...], preferred_element_type=jnp.float32)


def _mm_r2(aggp, deg2d, b, r, Wr):
    M, D = r.shape
    D2 = Wr.shape[1]
    blk = 1000
    Dh = D // 2
    return pl.pallas_call(
        _mm_r2_body,
        grid=(M // blk,),
        in_specs=[
            pl.BlockSpec((2, blk, Dh), lambda i: (0, i, 0)),
            pl.BlockSpec((blk, 1), lambda i: (i, 0)),
            pl.BlockSpec((1, D), lambda i: (0, 0)),
            pl.BlockSpec((blk, D), lambda i: (i, 0)),
            pl.BlockSpec((D, D2), lambda i: (0, 0)),
        ],
        out_specs=pl.BlockSpec((blk, D2), lambda i: (i, 0)),
        out_shape=jax.ShapeDtypeStruct((M, D2), jnp.float32),
    )(aggp, deg2d, b, r, Wr)


def _mm_b(aggp, deg2d, b, r, Wl):
    M, D = r.shape
    D2 = Wl.shape[1]
    blk = 1000
    Dh = D // 2
    return pl.pallas_call(
        _mm_b_body,
        grid=(M // blk,),
        in_specs=[
            pl.BlockSpec((2, blk, Dh), lambda i: (0, i, 0)),
            pl.BlockSpec((blk, 1), lambda i: (i, 0)),
            pl.BlockSpec((1, D), lambda i: (0, 0)),
            pl.BlockSpec((blk, D), lambda i: (i, 0)),
            pl.BlockSpec((D, D2), lambda i: (0, 0)),
        ],
        out_specs=pl.BlockSpec((2, blk, D2 // 2), lambda i: (0, i, 0)),
        out_shape=jax.ShapeDtypeStruct((2, NP, D2 // 2), jnp.float32),
    )(aggp, deg2d, b, r, Wl)


def _final_body(agg_ref, deg_ref, b_ref, r_ref, z_ref):
    agg = jnp.concatenate([agg_ref[0], agg_ref[1]], axis=-1)
    deg = deg_ref[:, 0]
    inv = 1.0 / jnp.maximum(deg, 1.0)
    z_ref[...] = agg * inv[:, None] + b_ref[...] + r_ref[...]


def _final(aggp, deg2d, b, r):
    M, D = r.shape
    blk = 1000
    Dh = D // 2
    return pl.pallas_call(
        _final_body,
        grid=(M // blk,),
        in_specs=[
            pl.BlockSpec((2, blk, Dh), lambda i: (0, i, 0)),
            pl.BlockSpec((blk, 1), lambda i: (i, 0)),
            pl.BlockSpec((1, D), lambda i: (0, 0)),
            pl.BlockSpec((blk, D), lambda i: (i, 0)),
        ],
        out_specs=pl.BlockSpec((blk, D), lambda i: (i, 0)),
        out_shape=jax.ShapeDtypeStruct((M, D), jnp.float32),
    )(aggp, deg2d, b, r)


def _make_sc_agg(Dh, with_deg):
    mesh = plsc.VectorSubcoreMesh(core_axis_name="c", subcore_axis_name="s")
    out_type = [jax.ShapeDtypeStruct((NC, NP, Dh), jnp.float32)]
    scratch = [
        pltpu.VMEM((CB + 1, CH), jnp.int32),
        pltpu.VMEM((CB + 1, CH), jnp.int32),
        pltpu.VMEM((2, G * CH, Dh), jnp.float32),
        pltpu.VMEM_SHARED((NP, Dh), jnp.float32),
        pltpu.VMEM_SHARED((NP, Dh), jnp.float32),
        pltpu.SemaphoreType.DMA,
        pltpu.SemaphoreType.DMA,
        pltpu.SemaphoreType.DMA,
    ]
    if with_deg:
        out_type.append(jax.ShapeDtypeStruct((NC, 1, NP), jnp.float32))
        scratch += [
            pltpu.VMEM((CH,), jnp.float32),
            pltpu.VMEM((PSUB,), jnp.float32),
            pltpu.VMEM_SHARED((NP,), jnp.float32),
            pltpu.SemaphoreType.DMA,
        ]

    @functools.partial(
        pl.kernel, mesh=mesh, out_type=out_type, scratch_types=scratch,
        compiler_params=pltpu.CompilerParams(use_tc_tiling_on_sc=False))
    def k(p_hbm, ei_hbm, *refs):
        if with_deg:
            (out_hbm, deg_hbm, src_v, dst_v, rows_v, acc_sh, tbl_sh,
             sem_i, sem_g, sem_s, ones_v, zero_v, deg_sh, sem_d) = refs
        else:
            (out_hbm, src_v, dst_v, rows_v, acc_sh, tbl_sh,
             sem_i, sem_g, sem_s) = refs
        c = lax.axis_index("c")
        s = lax.axis_index("s")
        base = s * CB + jnp.minimum(s, 4)
        extra = s < 4

        pltpu.async_copy(ei_hbm.at[0, pl.ds(base, CB)],
                         src_v.at[pl.ds(0, CB)], sem_i)
        pltpu.async_copy(ei_hbm.at[1, pl.ds(base, CB)],
                         dst_v.at[pl.ds(0, CB)], sem_i)
        pltpu.async_copy(p_hbm.at[c, pl.ds(s * PSUB, PSUB)],
                         tbl_sh.at[pl.ds(s * PSUB, PSUB)], sem_i)
        @pl.when(extra)
        def _():
            pltpu.async_copy(ei_hbm.at[0, pl.ds(base + CB, 1)],
                             src_v.at[pl.ds(CB, 1)], sem_i)
            pltpu.async_copy(ei_hbm.at[1, pl.ds(base + CB, 1)],
                             dst_v.at[pl.ds(CB, 1)], sem_i)

        def zrow(i, carry):
            for jj in range(Dh // 16):
                rows_v[0, i, pl.ds(jj * 16, 16)] = jnp.zeros((16,),
                                                             jnp.float32)
            return carry
        lax.fori_loop(0, CH, zrow, 0)
        for kk in range(PSUB // CH):
            pltpu.sync_copy(rows_v.at[0, pl.ds(0, CH)],
                            acc_sh.at[pl.ds(s * PSUB + kk * CH, CH)])
        if with_deg:
            def zdeg(i, carry):
                zero_v[pl.ds(i * 16, 16)] = jnp.zeros((16,), jnp.float32)
                return carry
            lax.fori_loop(0, PSUB // 16, zdeg, 0)
            for jj in range(CH // 16):
                ones_v[pl.ds(jj * 16, 16)] = jnp.ones((16,), jnp.float32)
            pltpu.sync_copy(zero_v, deg_sh.at[pl.ds(s * PSUB, PSUB)])
        pltpu.make_async_copy(ei_hbm.at[0, pl.ds(0, CB)],
                              src_v.at[pl.ds(0, CB)], sem_i).wait()
        pltpu.make_async_copy(ei_hbm.at[0, pl.ds(0, CB)],
                              dst_v.at[pl.ds(0, CB)], sem_i).wait()
        pltpu.make_async_copy(p_hbm.at[0, pl.ds(0, PSUB)],
                              tbl_sh.at[pl.ds(0, PSUB)], sem_i).wait()
        @pl.when(extra)
        def _():
            for _x in range(2):
                pltpu.make_async_copy(ei_hbm.at[0, pl.ds(0, 1)],
                                      src_v.at[pl.ds(CB, 1)], sem_i).wait()
        plsc.subcore_barrier()

        def g_start(ch, p, j):
            pltpu.async_copy(tbl_sh.at[src_v.at[ch]],
                             rows_v.at[p, pl.ds(j * CH, CH)], sem_g)

        def g_drain():
            pltpu.make_async_copy(tbl_sh.at[pl.ds(0, CH)],
                                  rows_v.at[0, pl.ds(0, CH)], sem_g).wait()

        def s_start(ch, p, j):
            pltpu.async_copy(rows_v.at[p, pl.ds(j * CH, CH)],
                             acc_sh.at[dst_v.at[ch]], sem_s, add=True)

        def s_drain():
            pltpu.make_async_copy(rows_v.at[0, pl.ds(0, CH)],
                                  acc_sh.at[pl.ds(0, CH)], sem_s).wait()

        def d_start(ch):
            pltpu.async_copy(ones_v, deg_sh.at[dst_v.at[ch]], sem_d,
                             add=True)

        def d_drain():
            pltpu.make_async_copy(ones_v, deg_sh.at[pl.ds(0, CH)],
                                  sem_d).wait()

        def fire_deg(n):
            return (c == 0) == (n < NG2)

        for j in range(G):
            g_start(j, 0, j)

        def grp(n, carry):
            p = lax.rem(n, 2)
            for j in range(G):
                g_drain()
            @pl.when(n >= 1)
            def _():
                for j in range(G):
                    s_drain()
            if with_deg:
                @pl.when((n >= 1) & fire_deg(n - 1))
                def _():
                    for j in range(G):
                        d_drain()
            @pl.when(n + 1 < NG)
            def _():
                for j in range(G):
                    g_start((n + 1) * G + j, 1 - p, j)
            for j in range(G):
                s_start(n * G + j, p, j)
            if with_deg:
                @pl.when(fire_deg(n))
                def _():
                    for j in range(G):
                        d_start(n * G + j)
            return carry
        lax.fori_loop(0, NG, grp, 0)
        for j in range(G):
            s_drain()
        if with_deg:
            @pl.when(fire_deg(NG - 1))
            def _():
                for j in range(G):
                    d_drain()
        @pl.when(extra)
        def _():
            g_start(CB, 0, 0)
            g_drain()
            s_start(CB, 0, 0)
            s_drain()
        if with_deg:
            @pl.when(extra & (c == 0))
            def _():
                d_start(CB)
                d_drain()
        plsc.subcore_barrier()

        pltpu.sync_copy(acc_sh.at[pl.ds(s * PSUB, PSUB)],
                        out_hbm.at[c, pl.ds(s * PSUB, PSUB)])
        if with_deg:
            pltpu.sync_copy(deg_sh.at[pl.ds(s * PSUB, PSUB)],
                            deg_hbm.at[c, 0, pl.ds(s * PSUB, PSUB)])

    return k


_sc_agg_cache = {}


def _sc_agg_call(Dh, with_deg, *args):
    key = (Dh, with_deg)
    if key not in _sc_agg_cache:
        _sc_agg_cache[key] = _make_sc_agg(Dh, with_deg)
    return _sc_agg_cache[key](*args)


def _impl(x, edge_index, Wl1, bl1, Wr1, Wl2, bl2, Wr2):
    ei = edge_index.astype(jnp.int32).reshape(2, EC, CH)

    p1s = _mm_a(x, Wl1)
    agg1p, degp = _sc_agg_call(32, True, p1s, ei)
    r1 = _mm_r1(x, Wr1)
    deg2d = (degp[0] + degp[1]).reshape(NP, 1)
    p2s = _mm_b(agg1p, deg2d, bl1.reshape(1, -1), r1, Wl2)
    (agg2p,) = _sc_agg_call(16, False, p2s, ei)
    r2 = _mm_r2(agg1p, deg2d, bl1.reshape(1, -1), r1, Wr2)
    z = _final(agg2p, deg2d, bl2.reshape(1, -1), r2)
    return z


kernel = jax.jit(_impl)
